# speedup vs baseline: 1.4640x; 1.4640x over previous
import jax
import jax.numpy as jnp
from jax.experimental import pallas as pl


def _pq_body(x_ref, w_ref, o_ref):
    o_ref[...] = jnp.dot(x_ref[...], w_ref[...], preferred_element_type=jnp.float32)


def _pq_matmul(x, w):
    n, d = x.shape
    block = 4000
    return pl.pallas_call(
        _pq_body,
        grid=(n // block,),
        in_specs=[
            pl.BlockSpec((block, d), lambda i: (i, 0)),
            pl.BlockSpec(w.shape, lambda i: (0, 0)),
        ],
        out_specs=pl.BlockSpec((block, w.shape[1]), lambda i: (i, 0)),
        out_shape=jax.ShapeDtypeStruct((n, w.shape[1]), jnp.float32),
    )(x, w)


def kernel(x, edge_index, dist, W1, b1, W2, b2):
    n, d_feat = x.shape
    d_hid = W1.shape[1]
    src = edge_index[0]
    dst = edge_index[1]
    A = W1[0:d_feat]
    B = W1[d_feat : 2 * d_feat]
    C = W1[2 * d_feat :]
    w_pq = jnp.concatenate([A - B, B], axis=1)
    pq = _pq_matmul(x, w_pq)
    P = pq[:, :d_hid]
    Q = pq[:, d_hid:]
    dist_c = dist @ C + b1
    m = jax.ops.segment_max(Q[src] + dist_c, dst, num_segments=n)
    agg = jnp.where(jnp.isneginf(m), 0.0, P + m)
    return agg @ W2 + b2


# speedup vs baseline: 2.2881x; 1.5628x over previous
import functools

import jax
import jax.numpy as jnp
from jax import lax
from jax.experimental import pallas as pl
from jax.experimental.pallas import tpu as pltpu
from jax.experimental.pallas import tpu_sc as plsc

_LANES = 16


def _node_prep_body(x_ref, w_ref, p_ref, q_ref):
    xw = jnp.dot(x_ref[...], w_ref[...], preferred_element_type=jnp.float32)
    p_ref[...] = xw[:, :_LANES]
    q_ref[...] = xw[:, _LANES:]


def _node_prep(x, w_big, block):
    n, d = x.shape
    return pl.pallas_call(
        _node_prep_body,
        grid=(n // block,),
        in_specs=[
            pl.BlockSpec((block, d), lambda i: (i, 0)),
            pl.BlockSpec(w_big.shape, lambda i: (0, 0)),
        ],
        out_specs=[
            pl.BlockSpec((block, _LANES), lambda i: (i, 0)),
            pl.BlockSpec((block, _LANES), lambda i: (i, 0)),
        ],
        out_shape=[
            jax.ShapeDtypeStruct((n, _LANES), jnp.float32),
            jax.ShapeDtypeStruct((n, _LANES), jnp.float32),
        ],
    )(x, w_big)


def _edge_prep_body(dist_ref, srcf_ref, dstf_ref, c_ref, b_ref, rec_ref):
    r = jnp.dot(dist_ref[...], c_ref[...], preferred_element_type=jnp.float32)
    r = r + b_ref[...]
    rec_ref[...] = jnp.concatenate(
        [
            r[:, :10],
            srcf_ref[...],
            dstf_ref[...],
            jnp.zeros((r.shape[0], 4), jnp.float32),
        ],
        axis=1,
    )


def _edge_prep(dist, srcf, dstf, c16, b16, block):
    e = dist.shape[0]
    return pl.pallas_call(
        _edge_prep_body,
        grid=(e // block,),
        in_specs=[
            pl.BlockSpec((block, 2), lambda i: (i, 0)),
            pl.BlockSpec((block, 1), lambda i: (i, 0)),
            pl.BlockSpec((block, 1), lambda i: (i, 0)),
            pl.BlockSpec(c16.shape, lambda i: (0, 0)),
            pl.BlockSpec(b16.shape, lambda i: (0, 0)),
        ],
        out_specs=pl.BlockSpec((block, _LANES), lambda i: (i, 0)),
        out_shape=jax.ShapeDtypeStruct((e, _LANES), jnp.float32),
    )(dist, srcf, dstf, c16, b16)


def _epilogue_body(acc_ref, p_ref, w2_ref, b2_ref, o_ref):
    a = acc_ref[...][:, :10]
    p = p_ref[...][:, :10]
    m = jnp.where(jnp.isneginf(a), 0.0, a + p)
    o_ref[...] = jnp.dot(m, w2_ref[...], preferred_element_type=jnp.float32) + b2_ref[...]


def _epilogue(acc, p16, w2, b2row, block):
    n = acc.shape[0]
    d_out = w2.shape[1]
    return pl.pallas_call(
        _epilogue_body,
        grid=(n // block,),
        in_specs=[
            pl.BlockSpec((block, _LANES), lambda i: (i, 0)),
            pl.BlockSpec((block, _LANES), lambda i: (i, 0)),
            pl.BlockSpec(w2.shape, lambda i: (0, 0)),
            pl.BlockSpec(b2row.shape, lambda i: (0, 0)),
        ],
        out_specs=pl.BlockSpec((block, d_out), lambda i: (i, 0)),
        out_shape=jax.ShapeDtypeStruct((n, d_out), jnp.float32),
    )(acc, p16, w2, b2row)


def _make_sc_aggregate(n, e):
    info = plsc.get_sparse_core_info()
    nc, ns, lanes = info.num_cores, info.num_subcores, info.num_lanes
    nw = nc * ns
    assert lanes == _LANES and n % nw == 0
    npw = n // nw
    ch = 8000
    assert e % ch == 0
    nchunk = e // ch
    gb = 256

    mesh = plsc.VectorSubcoreMesh(core_axis_name="c", subcore_axis_name="s")

    @functools.partial(
        pl.kernel,
        mesh=mesh,
        compiler_params=pltpu.CompilerParams(
            needs_layout_passes=False, use_tc_tiling_on_sc=False
        ),
        out_type=jax.ShapeDtypeStruct((n * _LANES,), jnp.float32),
        scratch_types=[
            pltpu.VMEM((npw * _LANES,), jnp.float32),
            pltpu.VMEM((ch,), jnp.int32),
            pltpu.VMEM((ch + _LANES,), jnp.int32),
            pltpu.VMEM((gb, _LANES), jnp.float32),
            pltpu.VMEM((gb, _LANES), jnp.float32),
            pltpu.VMEM((gb,), jnp.int32),
            pltpu.VMEM((gb,), jnp.int32),
            pltpu.SemaphoreType.DMA,
        ],
    )
    def sc_agg(dst_hbm, rec_hbm, q_hbm, acc_hbm,
               acc_v, dst_v, cid_v, rec_v, qrow_v, srcb_v, dstl_v, sem):
        wid = lax.axis_index("s") * nc + lax.axis_index("c")
        node_lo = wid * npw
        iota = lax.iota(jnp.int32, _LANES)
        ninf = jnp.full((_LANES,), -jnp.inf, jnp.float32)

        def init_body(r, _):
            plsc.store_scatter(acc_v, [r * _LANES + iota], ninf)
            return 0
        lax.fori_loop(0, npw, init_body, 0)

        def cinit_body(r, _):
            plsc.store_scatter(cid_v, [r * _LANES + iota],
                               jnp.full((_LANES,), wid, jnp.int32))
            return 0
        lax.fori_loop(0, (ch + _LANES) // _LANES, cinit_body, 0)

        def accum_batch(b, k):
            pltpu.async_copy(
                rec_hbm.at[cid_v.at[pl.ds(b * gb, gb)]], rec_v, sem
            ).wait()
            valid = k - b * gb
            for g in range(gb // _LANES):
                rows = g * _LANES + iota
                srcf = plsc.load_gather(
                    rec_v, [rows, jnp.full((_LANES,), 10, jnp.int32)])
                srci = plsc.bitcast(srcf, jnp.int32)
                srci = jnp.where(rows < valid, srci,
                                 jnp.full((_LANES,), node_lo, jnp.int32))
                srcb_v[g * _LANES:(g + 1) * _LANES] = srci
                dstf = plsc.load_gather(
                    rec_v, [rows, jnp.full((_LANES,), 11, jnp.int32)])
                dsti = plsc.bitcast(dstf, jnp.int32) - node_lo
                dsti = jnp.where(rows < valid, dsti,
                                 jnp.zeros((_LANES,), jnp.int32))
                dstl_v[g * _LANES:(g + 1) * _LANES] = dsti
            pltpu.async_copy(q_hbm.at[srcb_v], qrow_v, sem).wait()

            def edge_body(j, _):
                rows = jnp.full((_LANES,), j, jnp.int32)
                dvec = plsc.load_gather(dstl_v, [rows])
                rrow = plsc.load_gather(rec_v, [rows, iota])
                qrow = plsc.load_gather(qrow_v, [rows, iota])
                aidx = dvec * _LANES + iota
                arow = plsc.load_gather(acc_v, [aidx])
                plsc.store_scatter(acc_v, [aidx],
                                   jnp.maximum(arow, rrow + qrow))
                return 0
            nvalid = jnp.minimum(valid, gb)
            lax.fori_loop(0, nvalid, edge_body, 0)
            return k

        def chunk_body(c, _):
            base = c * ch
            pltpu.sync_copy(dst_hbm.at[pl.ds(base, ch)], dst_v)

            def scan_body(i, cur):
                dvec = plsc.load_gather(dst_v, [i * _LANES + iota])
                u = dvec - node_lo
                m = (u >= 0) & (u < npw)
                mi = m.astype(jnp.int32)
                pos = cur + plsc.cumsum(mi) - mi
                ids = base + i * _LANES + iota
                plsc.store_scatter(cid_v, [pos], ids, mask=m)
                return cur + jnp.sum(mi)

            k = lax.fori_loop(0, ch // _LANES, scan_body, 0)
            nb = (k + gb - 1) // gb
            lax.fori_loop(0, nb, accum_batch, k)
            return 0

        lax.fori_loop(0, nchunk, chunk_body, 0)

        pltpu.sync_copy(acc_v, acc_hbm.at[pl.ds(node_lo * _LANES, npw * _LANES)])

    return sc_agg


def kernel(x, edge_index, dist, W1, b1, W2, b2):
    n, d_feat = x.shape
    e = edge_index.shape[1]
    d_hid = W1.shape[1]
    src = edge_index[0]
    dst = edge_index[1]
    A = W1[0:d_feat]
    B = W1[d_feat:2 * d_feat]
    C = W1[2 * d_feat:]

    zpad = jnp.zeros((d_feat, _LANES - d_hid), jnp.float32)
    w_big = jnp.concatenate([A - B, zpad, B, zpad], axis=1)
    p16, q16 = _node_prep(x, w_big, block=4000)

    c16 = jnp.concatenate([C, jnp.zeros((2, _LANES - d_hid), jnp.float32)], axis=1)
    b16 = jnp.concatenate([b1, jnp.zeros((_LANES - d_hid,), jnp.float32)]).reshape(1, _LANES)
    srcf = lax.bitcast_convert_type(src, jnp.float32).reshape(e, 1)
    dstf = lax.bitcast_convert_type(dst, jnp.float32).reshape(e, 1)
    rec = _edge_prep(dist, srcf, dstf, c16, b16, block=8000)

    accf = _make_sc_aggregate(n, e)(dst, rec, q16)
    acc = accf.reshape(n, _LANES)

    return _epilogue(acc, p16, W2, b2.reshape(1, -1), block=4000)


# speedup vs baseline: 2.9508x; 1.2896x over previous
import functools

import jax
import jax.numpy as jnp
from jax import lax
from jax.experimental import pallas as pl
from jax.experimental.pallas import tpu as pltpu
from jax.experimental.pallas import tpu_sc as plsc

_LANES = 16


def _node_prep_body(x_ref, w_ref, p_ref, q_ref):
    xw = jnp.dot(x_ref[...], w_ref[...], preferred_element_type=jnp.float32)
    p_ref[...] = xw[:, :_LANES]
    q_ref[...] = xw[:, _LANES:]


def _node_prep(x, w_big, block):
    n, d = x.shape
    return pl.pallas_call(
        _node_prep_body,
        grid=(n // block,),
        in_specs=[
            pl.BlockSpec((block, d), lambda i: (i, 0)),
            pl.BlockSpec(w_big.shape, lambda i: (0, 0)),
        ],
        out_specs=[
            pl.BlockSpec((block, _LANES), lambda i: (i, 0)),
            pl.BlockSpec((block, _LANES), lambda i: (i, 0)),
        ],
        out_shape=[
            jax.ShapeDtypeStruct((n, _LANES), jnp.float32),
            jax.ShapeDtypeStruct((n, _LANES), jnp.float32),
        ],
    )(x, w_big)


def _edge_prep_body(dist_ref, srcf_ref, dstf_ref, c_ref, b_ref, rec_ref):
    r = jnp.dot(dist_ref[...], c_ref[...], preferred_element_type=jnp.float32)
    r = r + b_ref[...]
    rec_ref[...] = jnp.concatenate(
        [
            r[:, :10],
            srcf_ref[...],
            dstf_ref[...],
            jnp.zeros((r.shape[0], 4), jnp.float32),
        ],
        axis=1,
    )


def _edge_prep(dist, srcf, dstf, c16, b16, block):
    e = dist.shape[0]
    return pl.pallas_call(
        _edge_prep_body,
        grid=(e // block,),
        in_specs=[
            pl.BlockSpec((block, 2), lambda i: (i, 0)),
            pl.BlockSpec((block, 1), lambda i: (i, 0)),
            pl.BlockSpec((block, 1), lambda i: (i, 0)),
            pl.BlockSpec(c16.shape, lambda i: (0, 0)),
            pl.BlockSpec(b16.shape, lambda i: (0, 0)),
        ],
        out_specs=pl.BlockSpec((block, _LANES), lambda i: (i, 0)),
        out_shape=jax.ShapeDtypeStruct((e, _LANES), jnp.float32),
    )(dist, srcf, dstf, c16, b16)


def _epilogue_body(a0_ref, a1_ref, p_ref, w2_ref, b2_ref, o_ref):
    a = jnp.maximum(a0_ref[...][:, :10], a1_ref[...][:, :10])
    p = p_ref[...][:, :10]
    m = jnp.where(jnp.isneginf(a), 0.0, a + p)
    o_ref[...] = jnp.dot(m, w2_ref[...], preferred_element_type=jnp.float32) + b2_ref[...]


def _epilogue(acc2, p16, w2, b2row, block):
    n = p16.shape[0]
    nblk = n // block
    d_out = w2.shape[1]
    return pl.pallas_call(
        _epilogue_body,
        grid=(nblk,),
        in_specs=[
            pl.BlockSpec((block, _LANES), lambda i: (i, 0)),
            pl.BlockSpec((block, _LANES), lambda i: (i + nblk, 0)),
            pl.BlockSpec((block, _LANES), lambda i: (i, 0)),
            pl.BlockSpec(w2.shape, lambda i: (0, 0)),
            pl.BlockSpec(b2row.shape, lambda i: (0, 0)),
        ],
        out_specs=pl.BlockSpec((block, d_out), lambda i: (i, 0)),
        out_shape=jax.ShapeDtypeStruct((n, d_out), jnp.float32),
    )(acc2, acc2, p16, w2, b2row)


def _make_sc_aggregate(n, e):
    info = plsc.get_sparse_core_info()
    nc, ns, lanes = info.num_cores, info.num_subcores, info.num_lanes
    assert lanes == _LANES and n % ns == 0 and e % nc == 0
    npw = n // ns
    half = e // nc
    ch = 8000
    unroll = 4
    assert half % ch == 0 and (ch // _LANES) % unroll == 0
    nchunk = half // ch
    gb = 256

    mesh = plsc.VectorSubcoreMesh(core_axis_name="c", subcore_axis_name="s")

    @functools.partial(
        pl.kernel,
        mesh=mesh,
        compiler_params=pltpu.CompilerParams(
            needs_layout_passes=False, use_tc_tiling_on_sc=False
        ),
        out_type=jax.ShapeDtypeStruct((nc * n * _LANES,), jnp.float32),
        scratch_types=[
            pltpu.VMEM((npw * _LANES,), jnp.float32),
            pltpu.VMEM((ch,), jnp.int32),
            pltpu.VMEM((ch + _LANES,), jnp.int32),
            pltpu.VMEM((gb, _LANES), jnp.float32),
            pltpu.VMEM((gb, _LANES), jnp.float32),
            pltpu.VMEM((gb,), jnp.int32),
            pltpu.VMEM((gb,), jnp.int32),
            pltpu.SemaphoreType.DMA,
        ],
    )
    def sc_agg(dst_hbm, rec_hbm, q_hbm, acc_hbm,
               acc_v, dst_v, cid_v, rec_v, qrow_v, srcb_v, dstl_v, sem):
        cid = lax.axis_index("c")
        sid = lax.axis_index("s")
        node_lo = sid * npw
        iota = lax.iota(jnp.int32, _LANES)
        ninf = jnp.full((_LANES,), -jnp.inf, jnp.float32)

        def init_body(r, _):
            plsc.store_scatter(acc_v, [r * _LANES + iota], ninf)
            return 0
        lax.fori_loop(0, npw, init_body, 0)

        def cinit_body(r, _):
            plsc.store_scatter(cid_v, [r * _LANES + iota],
                               jnp.full((_LANES,), sid, jnp.int32))
            return 0
        lax.fori_loop(0, (ch + _LANES) // _LANES, cinit_body, 0)

        def accum_batch(b, k):
            pltpu.async_copy(
                rec_hbm.at[cid_v.at[pl.ds(b * gb, gb)]], rec_v, sem
            ).wait()
            valid = k - b * gb
            for g in range(gb // _LANES):
                rows = g * _LANES + iota
                srcf = plsc.load_gather(
                    rec_v, [rows, jnp.full((_LANES,), 10, jnp.int32)])
                srci = plsc.bitcast(srcf, jnp.int32)
                srci = jnp.where(rows < valid, srci,
                                 jnp.full((_LANES,), node_lo, jnp.int32))
                srcb_v[g * _LANES:(g + 1) * _LANES] = srci
                dstf = plsc.load_gather(
                    rec_v, [rows, jnp.full((_LANES,), 11, jnp.int32)])
                dsti = plsc.bitcast(dstf, jnp.int32) - node_lo
                dsti = jnp.where(rows < valid, dsti,
                                 jnp.zeros((_LANES,), jnp.int32))
                dstl_v[g * _LANES:(g + 1) * _LANES] = dsti
            pltpu.async_copy(q_hbm.at[srcb_v], qrow_v, sem).wait()

            def edge_body(j, _):
                rows = jnp.full((_LANES,), j, jnp.int32)
                dvec = plsc.load_gather(dstl_v, [rows])
                rrow = plsc.load_gather(rec_v, [rows, iota])
                qrow = plsc.load_gather(qrow_v, [rows, iota])
                aidx = dvec * _LANES + iota
                arow = plsc.load_gather(acc_v, [aidx])
                plsc.store_scatter(acc_v, [aidx],
                                   jnp.maximum(arow, rrow + qrow))
                return 0
            nvalid = jnp.minimum(valid, gb)
            lax.fori_loop(0, nvalid, edge_body, 0)
            return k

        def chunk_body(c, _):
            base = cid * half + c * ch
            pltpu.sync_copy(dst_hbm.at[pl.ds(base, ch)], dst_v)

            def scan_body(t, cur):
                i0 = t * unroll
                masks, cnts, idsl = [], [], []
                for u in range(unroll):
                    dvec = plsc.load_gather(dst_v, [(i0 + u) * _LANES + iota])
                    du = dvec - node_lo
                    m = (du >= 0) & (du < npw)
                    masks.append(m)
                    idsl.append(base + (i0 + u) * _LANES + iota)
                    cnts.append(jnp.sum(m.astype(jnp.int32)))
                cc = cur
                for u in range(unroll):
                    plsc.store_compressed(
                        cid_v.at[pl.ds(cc, _LANES)], idsl[u], mask=masks[u])
                    cc = cc + cnts[u]
                return cc

            k = lax.fori_loop(0, ch // _LANES // unroll, scan_body, 0)
            nb = (k + gb - 1) // gb
            lax.fori_loop(0, nb, accum_batch, k)
            return 0

        lax.fori_loop(0, nchunk, chunk_body, 0)

        out_off = (cid * n + node_lo) * _LANES
        pltpu.sync_copy(acc_v, acc_hbm.at[pl.ds(out_off, npw * _LANES)])

    return sc_agg


def kernel(x, edge_index, dist, W1, b1, W2, b2):
    n, d_feat = x.shape
    e = edge_index.shape[1]
    d_hid = W1.shape[1]
    src = edge_index[0]
    dst = edge_index[1]
    A = W1[0:d_feat]
    B = W1[d_feat:2 * d_feat]
    C = W1[2 * d_feat:]

    zpad = jnp.zeros((d_feat, _LANES - d_hid), jnp.float32)
    w_big = jnp.concatenate([A - B, zpad, B, zpad], axis=1)
    p16, q16 = _node_prep(x, w_big, block=4000)

    c16 = jnp.concatenate([C, jnp.zeros((2, _LANES - d_hid), jnp.float32)], axis=1)
    b16 = jnp.concatenate([b1, jnp.zeros((_LANES - d_hid,), jnp.float32)]).reshape(1, _LANES)
    srcf = lax.bitcast_convert_type(src, jnp.float32).reshape(e, 1)
    dstf = lax.bitcast_convert_type(dst, jnp.float32).reshape(e, 1)
    rec = _edge_prep(dist, srcf, dstf, c16, b16, block=8000)

    accf = _make_sc_aggregate(n, e)(dst, rec, q16)
    acc2 = accf.reshape(2 * n, _LANES)

    return _epilogue(acc2, p16, W2, b2.reshape(1, -1), block=4000)
